# baseline (speedup 1.0000x reference)
import jax
import jax.numpy as jnp
from jax import lax
from jax.experimental import pallas as pl
from jax.experimental.pallas import tpu as pltpu
from jax.experimental.pallas import tpu_sc as plsc

NC = 2
NS = 16
NW = NC * NS

f32 = jnp.float32


def _vector_mesh():
    return plsc.VectorSubcoreMesh(core_axis_name="c", subcore_axis_name="s")


def _deg_histogram(dst, ones_w, zeros_deg, n2, chunk):
    e = dst.shape[0]
    ew = e // NW
    nchunk = ew // chunk
    stripe = n2 // NS

    @pl.kernel(
        out_type=jax.ShapeDtypeStruct((NC, n2), f32),
        mesh=_vector_mesh(),
        scratch_types=[
            pltpu.VMEM((chunk,), jnp.int32),
            pltpu.VMEM((chunk,), f32),
            pltpu.VMEM_SHARED((n2,), f32),
        ],
    )
    def deg_kernel(dst_hbm, ones_hbm, zeros_hbm, out_hbm, idx_v, ones_v, acc):
        ci = lax.axis_index("c")
        si = lax.axis_index("s")
        wid = si * NC + ci
        pltpu.sync_copy(zeros_hbm.at[pl.ds(si * stripe, stripe)],
                        acc.at[pl.ds(si * stripe, stripe)])
        pltpu.sync_copy(ones_hbm, ones_v)
        plsc.subcore_barrier()

        @pl.loop(0, nchunk)
        def _(k):
            base = wid * ew + k * chunk
            pltpu.sync_copy(dst_hbm.at[pl.ds(base, chunk)], idx_v)
            pltpu.sync_copy(ones_v, acc.at[idx_v], add=True)

        plsc.subcore_barrier()
        pltpu.sync_copy(acc.at[pl.ds(si * stripe, stripe)],
                        out_hbm.at[ci, pl.ds(si * stripe, stripe)])

    return deg_kernel(dst, ones_w, zeros_deg)


def _message_pass(y4, gidx2, sidx2, zeros_rows, n4, chunk):
    e = gidx2.shape[0]
    ew = e // NW
    nchunk = ew // chunk
    npair = nchunk // 2
    stripe = n4 // NS

    @pl.kernel(
        out_type=jax.ShapeDtypeStruct((NC, n4, 128), f32),
        mesh=_vector_mesh(),
        scratch_types=[
            pltpu.VMEM((ew,), jnp.int32),
            pltpu.VMEM((ew,), jnp.int32),
            pltpu.VMEM((chunk, 128), f32),
            pltpu.VMEM((chunk, 128), f32),
            pltpu.VMEM_SHARED((n4, 128), f32),
            pltpu.SemaphoreType.DMA,
            pltpu.SemaphoreType.DMA,
        ],
    )
    def msg_kernel(y_hbm, gidx_hbm, sidx_hbm, zeros_hbm, out_hbm,
                   idx_g, idx_s, rows0, rows1, acc, sem0, sem1):
        ci = lax.axis_index("c")
        si = lax.axis_index("s")
        wid = si * NC + ci
        pltpu.sync_copy(zeros_hbm.at[pl.ds(si * stripe, stripe)],
                        acc.at[pl.ds(si * stripe, stripe)])
        pltpu.sync_copy(gidx_hbm.at[pl.ds(wid * ew, ew)], idx_g)
        pltpu.sync_copy(sidx_hbm.at[pl.ds(wid * ew, ew)], idx_s)
        plsc.subcore_barrier()

        @pl.loop(0, npair)
        def _(j):
            b0 = (2 * j) * chunk
            b1 = b0 + chunk
            h0 = pltpu.async_copy(y_hbm.at[idx_g.at[pl.ds(b0, chunk)]],
                                  rows0, sem0)
            h1 = pltpu.async_copy(y_hbm.at[idx_g.at[pl.ds(b1, chunk)]],
                                  rows1, sem1)
            h0.wait()
            pltpu.sync_copy(rows0, acc.at[idx_s.at[pl.ds(b0, chunk)]], add=True)
            h1.wait()
            pltpu.sync_copy(rows1, acc.at[idx_s.at[pl.ds(b1, chunk)]], add=True)

        plsc.subcore_barrier()
        pltpu.sync_copy(acc.at[pl.ds(si * stripe, stripe)],
                        out_hbm.at[ci, pl.ds(si * stripe, stripe)])

    return msg_kernel(y4, gidx2, sidx2, zeros_rows)


def _positioned_block(yv, r, h):
    n2 = yv.shape[0]
    wide = jnp.concatenate([yv] * 4, axis=1)
    lane = lax.broadcasted_iota(jnp.int32, (n2, 4 * h), 1)
    return jnp.where((lane // h) == r, wide, 0.0)


def _tc_pre(x, w1, src, dst, n2):
    n, _ = x.shape
    h = w1.shape[1]
    shape = src.shape

    def body(x_ref, w1_ref, src_ref, dst_ref, xw_ref, gidx_ref, sidx_ref):
        d = dst_ref[...]
        gidx_ref[...] = src_ref[...] + n2 * (d & 3)
        sidx_ref[...] = d >> 2
        xw_ref[pl.ds(n, n2 - n), :] = jnp.zeros((n2 - n, h), f32)
        xw_ref[pl.ds(0, n), :] = jnp.dot(x_ref[...], w1_ref[...],
                                         preferred_element_type=f32,
                                         precision=lax.Precision.HIGHEST)

    return pl.pallas_call(
        body,
        out_shape=(jax.ShapeDtypeStruct((n2, h), f32),
                   jax.ShapeDtypeStruct(shape, jnp.int32),
                   jax.ShapeDtypeStruct(shape, jnp.int32)))(x, w1, src, dst)


def _tc_scale_table(degp, xw):
    n2, h = xw.shape

    nb = 8
    bn = n2 // nb

    def body(degp_ref, xw_ref, y_ref, dinv_ref, y4_ref):
        r = pl.program_id(0)
        deg = degp_ref[0, :] + degp_ref[1, :] + 1.0
        dinv = lax.rsqrt(deg)[:, None]
        dinv_ref[...] = dinv
        yv = xw_ref[...] * dinv
        y_ref[...] = yv
        y4_ref[...] = _positioned_block(yv, r, h)

    return pl.pallas_call(
        body,
        grid=(4, nb),
        in_specs=[pl.BlockSpec((NC, bn), lambda r, i: (0, i)),
                  pl.BlockSpec((bn, h), lambda r, i: (i, 0))],
        out_specs=(pl.BlockSpec((bn, h), lambda r, i: (i, 0)),
                   pl.BlockSpec((bn, 1), lambda r, i: (i, 0)),
                   pl.BlockSpec((bn, 4 * h), lambda r, i: (r * nb + i, 0))),
        out_shape=(jax.ShapeDtypeStruct((n2, h), f32),
                   jax.ShapeDtypeStruct((n2, 1), f32),
                   jax.ShapeDtypeStruct((4 * n2, 4 * h), f32)))(degp, xw)


def _tc_layer2(accp, y1, dinv, b1, w2):
    n2, h = y1.shape

    nb = 8
    bn = n2 // nb

    def body(accp_ref, y1_ref, dinv_ref, b1_ref, w2_ref, y2_ref, y4_ref):
        r = pl.program_id(0)
        agg = accp_ref[0] + accp_ref[1] + y1_ref[...]
        hid = jnp.maximum(agg * dinv_ref[...] + b1_ref[...], 0.0)
        xw2 = jnp.dot(hid, w2_ref[...],
                      preferred_element_type=f32,
                      precision=lax.Precision.HIGHEST)
        yv = xw2 * dinv_ref[...]
        y2_ref[...] = yv
        y4_ref[...] = _positioned_block(yv, r, h)

    return pl.pallas_call(
        body,
        grid=(4, nb),
        in_specs=[pl.BlockSpec((NC, bn, h), lambda r, i: (0, i, 0)),
                  pl.BlockSpec((bn, h), lambda r, i: (i, 0)),
                  pl.BlockSpec((bn, 1), lambda r, i: (i, 0)),
                  pl.BlockSpec((1, h), lambda r, i: (0, 0)),
                  pl.BlockSpec((h, h), lambda r, i: (0, 0))],
        out_specs=(pl.BlockSpec((bn, h), lambda r, i: (i, 0)),
                   pl.BlockSpec((bn, 4 * h), lambda r, i: (r * nb + i, 0))),
        out_shape=(jax.ShapeDtypeStruct((n2, h), f32),
                   jax.ShapeDtypeStruct((4 * n2, 4 * h), f32)))(
            accp, y1, dinv, b1, w2)


def _tc_final(accp, y2, dinv, b2):
    n2, h = y2.shape

    def body(accp_ref, y2_ref, dinv_ref, b2_ref, o_ref):
        agg = accp_ref[0] + accp_ref[1] + y2_ref[...]
        o_ref[...] = agg * dinv_ref[...] + b2_ref[...]

    return pl.pallas_call(
        body, out_shape=jax.ShapeDtypeStruct((n2, h), f32))(
            accp, y2, dinv, b2)


def kernel(x, edge_index, W1, b1, W2, b2):
    n, _ = x.shape
    h = W1.shape[1]
    e = edge_index.shape[1]
    chunk = 2000
    mchunk = 200
    n2 = ((n + NW * 8 - 1) // (NW * 8)) * (NW * 8)
    n4 = n2 // 4

    src = edge_index[0]
    dst = edge_index[1]
    zeros_deg = jnp.zeros((n2,), f32)
    zeros_rows = jnp.zeros((n4, 128), f32)
    ones_w = jnp.ones((chunk,), f32)
    b1r = b1.reshape(1, h)
    b2r = b2.reshape(1, h)

    degp = _deg_histogram(dst, ones_w, zeros_deg, n2, chunk)
    xw1p, gidxw, sidxw = _tc_pre(x, W1, src.reshape(-1, 128),
                                 dst.reshape(-1, 128), n2)
    gidx, sidx = gidxw.reshape(e), sidxw.reshape(e)
    y1, dinv, y4_1 = _tc_scale_table(degp, xw1p)
    acc1 = _message_pass(y4_1, gidx, sidx, zeros_rows, n4, mchunk)
    accp1 = acc1.reshape(NC, n2, h)
    y2, y4_2 = _tc_layer2(accp1, y1, dinv, b1r, W2)
    acc2 = _message_pass(y4_2, gidx, sidx, zeros_rows, n4, mchunk)
    accp2 = acc2.reshape(NC, n2, h)
    out = _tc_final(accp2, y2, dinv, b2r)
    return out[:n]


# speedup vs baseline: 1.3671x; 1.3671x over previous
import jax
import jax.numpy as jnp
from jax import lax
from jax.experimental import pallas as pl
from jax.experimental.pallas import tpu as pltpu
from jax.experimental.pallas import tpu_sc as plsc

NC = 2
NS = 16
NW = NC * NS

f32 = jnp.float32


def _vector_mesh():
    return plsc.VectorSubcoreMesh(core_axis_name="c", subcore_axis_name="s")


def _deg_histogram(dst, ones_w, zeros_deg, n2, chunk):
    e = dst.shape[0]
    ew = e // NW
    nchunk = ew // chunk
    stripe = n2 // NS

    @pl.kernel(
        out_type=jax.ShapeDtypeStruct((NC, n2), f32),
        mesh=_vector_mesh(),
        scratch_types=[
            pltpu.VMEM((chunk,), jnp.int32),
            pltpu.VMEM((chunk,), f32),
            pltpu.VMEM_SHARED((n2,), f32),
        ],
    )
    def deg_kernel(dst_hbm, ones_hbm, zeros_hbm, out_hbm, idx_v, ones_v, acc):
        ci = lax.axis_index("c")
        si = lax.axis_index("s")
        wid = si * NC + ci
        pltpu.sync_copy(zeros_hbm.at[pl.ds(si * stripe, stripe)],
                        acc.at[pl.ds(si * stripe, stripe)])
        pltpu.sync_copy(ones_hbm, ones_v)
        plsc.subcore_barrier()

        @pl.loop(0, nchunk)
        def _(k):
            base = wid * ew + k * chunk
            pltpu.sync_copy(dst_hbm.at[pl.ds(base, chunk)], idx_v)
            pltpu.sync_copy(ones_v, acc.at[idx_v], add=True)

        plsc.subcore_barrier()
        pltpu.sync_copy(acc.at[pl.ds(si * stripe, stripe)],
                        out_hbm.at[ci, pl.ds(si * stripe, stripe)])

    return deg_kernel(dst, ones_w, zeros_deg)


def _message_pass(y4, gidx2, sidx2, zeros_rows, n4, chunk):
    e = gidx2.shape[0]
    ew = e // NW
    nchunk = ew // chunk
    npair = nchunk // 2
    stripe = n4 // NS

    @pl.kernel(
        out_type=jax.ShapeDtypeStruct((NC, n4, 128), f32),
        mesh=_vector_mesh(),
        scratch_types=[
            pltpu.VMEM((ew,), jnp.int32),
            pltpu.VMEM((ew,), jnp.int32),
            pltpu.VMEM((chunk, 128), f32),
            pltpu.VMEM((chunk, 128), f32),
            pltpu.VMEM_SHARED((n4, 128), f32),
            pltpu.SemaphoreType.DMA,
            pltpu.SemaphoreType.DMA,
        ],
    )
    def msg_kernel(y_hbm, gidx_hbm, sidx_hbm, zeros_hbm, out_hbm,
                   idx_g, idx_s, rows0, rows1, acc, sem0, sem1):
        ci = lax.axis_index("c")
        si = lax.axis_index("s")
        wid = si * NC + ci
        pltpu.sync_copy(zeros_hbm.at[pl.ds(si * stripe, stripe)],
                        acc.at[pl.ds(si * stripe, stripe)])
        pltpu.sync_copy(gidx_hbm.at[pl.ds(wid * ew, ew)], idx_g)
        pltpu.sync_copy(sidx_hbm.at[pl.ds(wid * ew, ew)], idx_s)
        plsc.subcore_barrier()

        pltpu.async_copy(y_hbm.at[idx_g.at[pl.ds(0, chunk)]], rows0, sem0)
        pltpu.async_copy(y_hbm.at[idx_g.at[pl.ds(chunk, chunk)]], rows1, sem1)

        @pl.loop(0, npair)
        def _(j):
            b0 = (2 * j) * chunk
            b1 = b0 + chunk
            n0 = (b0 + 2 * chunk) % ew
            n1 = (b1 + 2 * chunk) % ew
            pltpu.make_async_copy(y_hbm.at[idx_g.at[pl.ds(b0, chunk)]],
                                  rows0, sem0).wait()
            pltpu.sync_copy(rows0, acc.at[idx_s.at[pl.ds(b0, chunk)]], add=True)
            pltpu.async_copy(y_hbm.at[idx_g.at[pl.ds(n0, chunk)]], rows0, sem0)
            pltpu.make_async_copy(y_hbm.at[idx_g.at[pl.ds(b1, chunk)]],
                                  rows1, sem1).wait()
            pltpu.sync_copy(rows1, acc.at[idx_s.at[pl.ds(b1, chunk)]], add=True)
            pltpu.async_copy(y_hbm.at[idx_g.at[pl.ds(n1, chunk)]], rows1, sem1)

        pltpu.make_async_copy(y_hbm.at[idx_g.at[pl.ds(0, chunk)]],
                              rows0, sem0).wait()
        pltpu.make_async_copy(y_hbm.at[idx_g.at[pl.ds(chunk, chunk)]],
                              rows1, sem1).wait()

        plsc.subcore_barrier()
        pltpu.sync_copy(acc.at[pl.ds(si * stripe, stripe)],
                        out_hbm.at[ci, pl.ds(si * stripe, stripe)])

    return msg_kernel(y4, gidx2, sidx2, zeros_rows)


def _tc_indices(src, dst, n2):
    shape = src.shape

    def body(src_ref, dst_ref, gidx_ref, sidx_ref):
        d = dst_ref[...]
        gidx_ref[...] = src_ref[...] + n2 * (d & 3)
        sidx_ref[...] = d >> 2

    return pl.pallas_call(
        body,
        out_shape=(jax.ShapeDtypeStruct(shape, jnp.int32),
                   jax.ShapeDtypeStruct(shape, jnp.int32)))(src, dst)


def _tc_build_table(y, n2, h):
    def body(y_ref, y4_ref):
        yv = y_ref[...]
        z = jnp.zeros_like(yv)
        for r in range(4):
            blocks = [z] * 4
            blocks[r] = yv
            y4_ref[pl.ds(r * n2, n2), :] = jnp.concatenate(blocks, axis=1)

    return pl.pallas_call(
        body, out_shape=jax.ShapeDtypeStruct((4 * n2, 4 * h), f32))(y)


def _tc_xw(x, w):
    n, _ = x.shape
    h = w.shape[1]

    def body(x_ref, w_ref, o_ref):
        o_ref[...] = jnp.dot(x_ref[...], w_ref[...],
                             preferred_element_type=f32,
                             precision=lax.Precision.HIGHEST)

    return pl.pallas_call(
        body, out_shape=jax.ShapeDtypeStruct((n, h), f32))(x, w)


def _tc_scale(degp, xw):
    n2, h = xw.shape

    def body(degp_ref, xw_ref, y_ref, dinv_ref):
        deg = degp_ref[0, :] + degp_ref[1, :] + 1.0
        dinv = lax.rsqrt(deg)[:, None]
        dinv_ref[...] = dinv
        y_ref[...] = xw_ref[...] * dinv

    return pl.pallas_call(
        body,
        out_shape=(jax.ShapeDtypeStruct((n2, h), f32),
                   jax.ShapeDtypeStruct((n2, 1), f32)))(degp, xw)


def _tc_layer2_in(accp, y1, dinv, b1, w2):
    n2, h = y1.shape

    def body(accp_ref, y1_ref, dinv_ref, b1_ref, w2_ref, y2_ref):
        agg = accp_ref[0] + accp_ref[1] + y1_ref[...]
        hid = jnp.maximum(agg * dinv_ref[...] + b1_ref[...], 0.0)
        xw2 = jnp.dot(hid, w2_ref[...],
                      preferred_element_type=f32,
                      precision=lax.Precision.HIGHEST)
        y2_ref[...] = xw2 * dinv_ref[...]

    return pl.pallas_call(
        body, out_shape=jax.ShapeDtypeStruct((n2, h), f32))(
            accp, y1, dinv, b1, w2)


def _tc_final(accp, y2, dinv, b2):
    n2, h = y2.shape

    def body(accp_ref, y2_ref, dinv_ref, b2_ref, o_ref):
        agg = accp_ref[0] + accp_ref[1] + y2_ref[...]
        o_ref[...] = agg * dinv_ref[...] + b2_ref[...]

    return pl.pallas_call(
        body, out_shape=jax.ShapeDtypeStruct((n2, h), f32))(
            accp, y2, dinv, b2)


def kernel(x, edge_index, W1, b1, W2, b2):
    n, _ = x.shape
    h = W1.shape[1]
    e = edge_index.shape[1]
    chunk = 2000
    mchunk = 200
    n2 = ((n + NW * 8 - 1) // (NW * 8)) * (NW * 8)
    n4 = n2 // 4

    src = edge_index[0]
    dst = edge_index[1]
    zeros_deg = jnp.zeros((n2,), f32)
    zeros_rows = jnp.zeros((n4, 128), f32)
    ones_w = jnp.ones((chunk,), f32)
    b1r = b1.reshape(1, h)
    b2r = b2.reshape(1, h)

    degp = _deg_histogram(dst, ones_w, zeros_deg, n2, chunk)
    gidxw, sidxw = _tc_indices(src.reshape(-1, 128),
                               dst.reshape(-1, 128), n2)
    gidx, sidx = gidxw.reshape(e), sidxw.reshape(e)
    xw1 = _tc_xw(x, W1)
    xw1p = jnp.pad(xw1, ((0, n2 - n), (0, 0)))
    y1, dinv = _tc_scale(degp, xw1p)
    y4_1 = _tc_build_table(y1, n2, h)
    acc1 = _message_pass(y4_1, gidx, sidx, zeros_rows, n4, mchunk)
    accp1 = acc1.reshape(NC, n2, h)
    y2 = _tc_layer2_in(accp1, y1, dinv, b1r, W2)
    y4_2 = _tc_build_table(y2, n2, h)
    acc2 = _message_pass(y4_2, gidx, sidx, zeros_rows, n4, mchunk)
    accp2 = acc2.reshape(NC, n2, h)
    out = _tc_final(accp2, y2, dinv, b2r)
    return out[:n]
